# baseline (speedup 1.0000x reference)
import functools

import jax
import jax.numpy as jnp
from jax import lax
from jax.experimental import pallas as pl
from jax.experimental.pallas import tpu as pltpu
from jax.experimental.pallas import tpu_sc as plsc

N = 10000
E = 320000
D = 128
DH = D // 2
CAT = 4 * D

NC = 2
NS = 16
NW = NC * NS

CH = 128
K = 8
KB = 4
E2 = 327680
RPW = E2 // (NW * CH)
RPT = E2 // (NS * CH)
GROUPS_D = RPW // K
NGRP = RPT // KB

NPAD = 10112
LT = NPAD // NS
WCH = 104
WREM = LT - 6 * WCH
DEGW = 8

BR = 1000


def _sc_mesh():
    return plsc.VectorSubcoreMesh(
        core_axis_name="c", subcore_axis_name="s", num_cores=NC, num_subcores=NS
    )


@functools.cache
def _sc_degree_kernel():
    return functools.partial(
        pl.kernel,
        out_type=jax.ShapeDtypeStruct((NC, N, DEGW), jnp.float32),
        mesh=_sc_mesh(),
        compiler_params=pltpu.CompilerParams(use_tc_tiling_on_sc=False),
        scratch_types=[
            pltpu.VMEM((K, CH), jnp.int32),
            pltpu.VMEM((CH, DEGW), jnp.float32),
            pltpu.VMEM((CH, DEGW), jnp.float32),
            pltpu.VMEM_SHARED((NPAD, DEGW), jnp.float32),
            pltpu.SemaphoreType.DMA,
        ],
    )(_sc_degree_body)


def _sc_degree_body(const_hbm, dst2_hbm, out_hbm, didx, ones, stage, acc, sem):
    cid = lax.axis_index("c")
    sid = lax.axis_index("s")
    eid = cid * NS + sid

    pltpu.sync_copy(const_hbm.at[0], ones)
    pltpu.sync_copy(const_hbm.at[1], stage)

    def zero_chunk(z, _):
        pltpu.sync_copy(stage.at[pl.ds(0, WCH), :],
                        acc.at[pl.ds(sid * LT + z * WCH, WCH), :])
        return 0

    lax.fori_loop(0, LT // WCH, zero_chunk, 0)
    pltpu.sync_copy(stage.at[pl.ds(0, WREM), :],
                    acc.at[pl.ds(sid * LT + (LT // WCH) * WCH, WREM), :])
    plsc.subcore_barrier()

    def group(g, _):
        row0 = eid * RPW + g * K
        pltpu.sync_copy(dst2_hbm.at[pl.ds(row0, K), :], didx)
        cps = [
            pltpu.async_copy(ones, acc.at[didx.at[k]], sem, add=True)
            for k in range(K)
        ]
        for c in cps:
            c.wait()
        return 0

    lax.fori_loop(0, GROUPS_D, group, 0)
    plsc.subcore_barrier()

    nch = lax.select(sid == NS - 1, (N - (NS - 1) * LT) // WCH, LT // WCH)

    def write_chunk(z, _):
        r0 = sid * LT + z * WCH
        pltpu.sync_copy(acc.at[pl.ds(r0, WCH), :], stage.at[pl.ds(0, WCH), :])
        pltpu.sync_copy(stage.at[pl.ds(0, WCH), :], out_hbm.at[cid, pl.ds(r0, WCH), :])
        return 0

    lax.fori_loop(0, nch, write_chunk, 0)

    @pl.when(sid < NS - 1)
    def _():
        r0 = sid * LT + 6 * WCH
        pltpu.sync_copy(acc.at[pl.ds(r0, WREM), :], stage.at[pl.ds(0, WREM), :])
        pltpu.sync_copy(stage.at[pl.ds(0, WREM), :], out_hbm.at[cid, pl.ds(r0, WREM), :])


@functools.cache
def _sc_aggregate_kernel():
    return functools.partial(
        pl.kernel,
        out_type=jax.ShapeDtypeStruct((NC, N, DH), jnp.float32),
        mesh=_sc_mesh(),
        compiler_params=pltpu.CompilerParams(use_tc_tiling_on_sc=False),
        scratch_types=[
            pltpu.VMEM((K, CH), jnp.int32),
            pltpu.VMEM((K, CH), jnp.int32),
            pltpu.VMEM((KB, CH, DH), jnp.float32),
            pltpu.VMEM((KB, CH, DH), jnp.float32),
            pltpu.VMEM((CH, DH), jnp.float32),
            pltpu.VMEM_SHARED((NPAD, DH), jnp.float32),
            pltpu.SemaphoreType.DMA,
        ],
    )(_sc_aggregate_body)


def _sc_aggregate_body(mpa_hbm, mpb_hbm, src2_hbm, dst2_hbm, out_hbm,
                       sidx, didx, rows_a, rows_b, stage, acc, sem):
    cid = lax.axis_index("c")
    sid = lax.axis_index("s")

    def zero_row(r, _):
        def zero_col(c, _):
            stage[r, pl.ds(c * 16, 16)] = jnp.zeros((16,), jnp.float32)
            return 0

        lax.fori_loop(0, DH // 16, zero_col, 0)
        return 0

    lax.fori_loop(0, CH, zero_row, 0)

    def zero_chunk(z, _):
        pltpu.sync_copy(stage.at[pl.ds(0, WCH), :],
                        acc.at[pl.ds(sid * LT + z * WCH, WCH), :])
        return 0

    lax.fori_loop(0, LT // WCH, zero_chunk, 0)
    pltpu.sync_copy(stage.at[pl.ds(0, WREM), :],
                    acc.at[pl.ds(sid * LT + (LT // WCH) * WCH, WREM), :])
    plsc.subcore_barrier()

    def make_pipeline(mp_hbm):
        NBODY = RPT // K

        def run():
            def body(t, _):
                row0 = sid * RPT + t * K
                pltpu.sync_copy(src2_hbm.at[pl.ds(row0, K), :], sidx)
                pltpu.sync_copy(dst2_hbm.at[pl.ds(row0, K), :], didx)
                for half in range(K // KB):
                    rows = rows_a if half == 0 else rows_b
                    gcps = [
                        pltpu.async_copy(mp_hbm.at[sidx.at[half * KB + k]], rows.at[k], sem)
                        for k in range(KB)
                    ]
                    for c in gcps:
                        c.wait()
                    scps = [
                        pltpu.async_copy(rows.at[k], acc.at[didx.at[half * KB + k]], sem, add=True)
                        for k in range(KB)
                    ]
                    for c in scps:
                        c.wait()
                return 0

            lax.fori_loop(0, NBODY, body, 0)

        return run

    @pl.when(cid == 0)
    def _():
        make_pipeline(mpa_hbm)()

    @pl.when(cid == 1)
    def _():
        make_pipeline(mpb_hbm)()

    plsc.subcore_barrier()

    nch = lax.select(sid == NS - 1, (N - (NS - 1) * LT) // WCH, LT // WCH)

    def write_chunk(z, _):
        r0 = sid * LT + z * WCH
        pltpu.sync_copy(acc.at[pl.ds(r0, WCH), :], stage.at[pl.ds(0, WCH), :])
        pltpu.sync_copy(stage.at[pl.ds(0, WCH), :], out_hbm.at[cid, pl.ds(r0, WCH), :])
        return 0

    lax.fori_loop(0, nch, write_chunk, 0)

    @pl.when(sid < NS - 1)
    def _():
        r0 = sid * LT + 6 * WCH
        pltpu.sync_copy(acc.at[pl.ds(r0, WREM), :], stage.at[pl.ds(0, WREM), :])
        pltpu.sync_copy(stage.at[pl.ds(0, WREM), :], out_hbm.at[cid, pl.ds(r0, WREM), :])


def _mm(x, w):
    def body(x_ref, w_ref, o_ref):
        o_ref[...] = jnp.dot(x_ref[...], w_ref[...], preferred_element_type=jnp.float32)

    return pl.pallas_call(
        body,
        grid=(N // BR,),
        in_specs=[
            pl.BlockSpec((BR, D), lambda i: (i, 0)),
            pl.BlockSpec((D, D), lambda i: (0, 0)),
        ],
        out_specs=pl.BlockSpec((BR, D), lambda i: (i, 0)),
        out_shape=jax.ShapeDtypeStruct((N, D), jnp.float32),
    )(x, w)


def _dinv_scale(degp, u1):
    def body(dg_ref, u_ref, dv_ref, ma_ref, mb_ref):
        dv = lax.rsqrt(dg_ref[0] + dg_ref[1] + 1.0)
        dv_ref[...] = dv
        m = u_ref[...] * dv[:, 0:1]
        ma_ref[...] = m[:, :DH]
        mb_ref[...] = m[:, DH:]

    return pl.pallas_call(
        body,
        grid=(N // BR,),
        in_specs=[
            pl.BlockSpec((NC, BR, DEGW), lambda i: (0, i, 0)),
            pl.BlockSpec((BR, D), lambda i: (i, 0)),
        ],
        out_specs=[
            pl.BlockSpec((BR, DEGW), lambda i: (i, 0)),
            pl.BlockSpec((BR, DH), lambda i: (i, 0)),
            pl.BlockSpec((BR, DH), lambda i: (i, 0)),
        ],
        out_shape=[
            jax.ShapeDtypeStruct((N, DEGW), jnp.float32),
            jax.ShapeDtypeStruct((N, DH), jnp.float32),
            jax.ShapeDtypeStruct((N, DH), jnp.float32),
        ],
    )(degp, u1)


def _mid(aggp, ma, mb, dinv, b, w):
    def body(a_ref, ma_ref, mb_ref, dv_ref, b_ref, w_ref, h_ref, mna_ref, mnb_ref):
        dv = dv_ref[:, 0:1]
        agg = jnp.concatenate([a_ref[0] + ma_ref[...], a_ref[1] + mb_ref[...]], axis=1)
        h = dv * agg + b_ref[...]
        h_ref[...] = h
        mn = jnp.dot(dv * h, w_ref[...], preferred_element_type=jnp.float32)
        mna_ref[...] = mn[:, :DH]
        mnb_ref[...] = mn[:, DH:]

    return pl.pallas_call(
        body,
        grid=(N // BR,),
        in_specs=[
            pl.BlockSpec((NC, BR, DH), lambda i: (0, i, 0)),
            pl.BlockSpec((BR, DH), lambda i: (i, 0)),
            pl.BlockSpec((BR, DH), lambda i: (i, 0)),
            pl.BlockSpec((BR, DEGW), lambda i: (i, 0)),
            pl.BlockSpec((1, D), lambda i: (0, 0)),
            pl.BlockSpec((D, D), lambda i: (0, 0)),
        ],
        out_specs=[
            pl.BlockSpec((BR, D), lambda i: (i, 0)),
            pl.BlockSpec((BR, DH), lambda i: (i, 0)),
            pl.BlockSpec((BR, DH), lambda i: (i, 0)),
        ],
        out_shape=[
            jax.ShapeDtypeStruct((N, D), jnp.float32),
            jax.ShapeDtypeStruct((N, DH), jnp.float32),
            jax.ShapeDtypeStruct((N, DH), jnp.float32),
        ],
    )(aggp, ma, mb, dinv, b, w)


def _final(x, hs, wp, bp):
    def body(x_ref, hs_ref, wp_ref, bp_ref, o_ref):
        acc = jnp.dot(x_ref[...], wp_ref[0:D], preferred_element_type=jnp.float32)
        acc = acc + jnp.dot(hs_ref[0], wp_ref[D:2 * D], preferred_element_type=jnp.float32)
        acc = acc + jnp.dot(hs_ref[1], wp_ref[2 * D:3 * D], preferred_element_type=jnp.float32)
        acc = acc + jnp.dot(hs_ref[2], wp_ref[3 * D:4 * D], preferred_element_type=jnp.float32)
        o_ref[...] = acc + bp_ref[...]

    return pl.pallas_call(
        body,
        grid=(N // BR,),
        in_specs=[
            pl.BlockSpec((BR, D), lambda i: (i, 0)),
            pl.BlockSpec((3, BR, D), lambda i: (0, i, 0)),
            pl.BlockSpec((CAT, D), lambda i: (0, 0)),
            pl.BlockSpec((1, D), lambda i: (0, 0)),
        ],
        out_specs=pl.BlockSpec((BR, D), lambda i: (i, 0)),
        out_shape=jax.ShapeDtypeStruct((N, D), jnp.float32),
    )(x, hs, wp, bp)


def kernel(x, edge_index, W1, b1, W2, b2, W3, b3, Wp, bp):
    pad = E2 - E
    pad_ar = jnp.arange(pad, dtype=jnp.int32)
    src = jnp.concatenate([edge_index[0], pad_ar % N])
    dst = jnp.concatenate([edge_index[1], N + pad_ar % (NPAD - N)])
    src2 = src.reshape(E2 // CH, CH)
    dst2 = dst.reshape(E2 // CH, CH)
    b1r = b1.reshape(1, D)
    b2r = b2.reshape(1, D)
    b3r = b3.reshape(1, D)
    bpr = bp.reshape(1, D)

    sc_degree = _sc_degree_kernel()
    sc_aggregate = _sc_aggregate_kernel()

    const = jnp.stack([jnp.ones((CH, DEGW), jnp.float32),
                       jnp.zeros((CH, DEGW), jnp.float32)])
    degp = sc_degree(const, dst2)
    u1 = _mm(x, W1)
    dinv, m1a, m1b = _dinv_scale(degp, u1)

    wnext = jnp.stack([W2, W3, jnp.zeros_like(W3)])
    bstack = jnp.stack([b1r, b2r, b3r])

    def step(carry, xs):
        ma, mb = carry
        wn, bk = xs
        a = sc_aggregate(ma, mb, src2, dst2)
        h, mna, mnb = _mid(a, ma, mb, dinv, bk, wn)
        return (mna, mnb), h

    _, hs = lax.scan(step, (m1a, m1b), (wnext, bstack))
    return _final(x, hs, Wp, bpr)


# speedup vs baseline: 1.0803x; 1.0803x over previous
import functools

import jax
import jax.numpy as jnp
from jax import lax
from jax.experimental import pallas as pl
from jax.experimental.pallas import tpu as pltpu
from jax.experimental.pallas import tpu_sc as plsc

N = 10000
E = 320000
D = 128
DH = D // 2
CAT = 4 * D

NC = 2
NS = 16
NW = NC * NS

CH = 128
K = 8
KB = 4
E2 = 327680
RPW = E2 // (NW * CH)
RPT = E2 // (NS * CH)
GROUPS_D = RPW // K
NGRP = RPT // KB

NPAD = 10112
LT = NPAD // NS
WCH = 104
WREM = LT - 6 * WCH
DEGW = 8

BR = 1000


def _sc_mesh():
    return plsc.VectorSubcoreMesh(
        core_axis_name="c", subcore_axis_name="s", num_cores=NC, num_subcores=NS
    )


@functools.cache
def _sc_degree_kernel():
    return functools.partial(
        pl.kernel,
        out_type=jax.ShapeDtypeStruct((NC, N, DEGW), jnp.float32),
        mesh=_sc_mesh(),
        compiler_params=pltpu.CompilerParams(use_tc_tiling_on_sc=False),
        scratch_types=[
            pltpu.VMEM((K, CH), jnp.int32),
            pltpu.VMEM((CH, DEGW), jnp.float32),
            pltpu.VMEM((CH, DEGW), jnp.float32),
            pltpu.VMEM_SHARED((NPAD, DEGW), jnp.float32),
            pltpu.SemaphoreType.DMA,
        ],
    )(_sc_degree_body)


def _sc_degree_body(const_hbm, dst2_hbm, out_hbm, didx, ones, stage, acc, sem):
    cid = lax.axis_index("c")
    sid = lax.axis_index("s")
    eid = cid * NS + sid

    pltpu.sync_copy(const_hbm.at[0], ones)
    pltpu.sync_copy(const_hbm.at[1], stage)

    def zero_chunk(z, _):
        pltpu.sync_copy(stage.at[pl.ds(0, WCH), :],
                        acc.at[pl.ds(sid * LT + z * WCH, WCH), :])
        return 0

    lax.fori_loop(0, LT // WCH, zero_chunk, 0)
    pltpu.sync_copy(stage.at[pl.ds(0, WREM), :],
                    acc.at[pl.ds(sid * LT + (LT // WCH) * WCH, WREM), :])
    plsc.subcore_barrier()

    def group(g, _):
        row0 = eid * RPW + g * K
        pltpu.sync_copy(dst2_hbm.at[pl.ds(row0, K), :], didx)
        cps = [
            pltpu.async_copy(ones, acc.at[didx.at[k]], sem, add=True)
            for k in range(K)
        ]
        for c in cps:
            c.wait()
        return 0

    lax.fori_loop(0, GROUPS_D, group, 0)
    plsc.subcore_barrier()

    nch = lax.select(sid == NS - 1, (N - (NS - 1) * LT) // WCH, LT // WCH)

    def write_chunk(z, _):
        r0 = sid * LT + z * WCH
        pltpu.sync_copy(acc.at[pl.ds(r0, WCH), :], stage.at[pl.ds(0, WCH), :])
        pltpu.sync_copy(stage.at[pl.ds(0, WCH), :], out_hbm.at[cid, pl.ds(r0, WCH), :])
        return 0

    lax.fori_loop(0, nch, write_chunk, 0)

    @pl.when(sid < NS - 1)
    def _():
        r0 = sid * LT + 6 * WCH
        pltpu.sync_copy(acc.at[pl.ds(r0, WREM), :], stage.at[pl.ds(0, WREM), :])
        pltpu.sync_copy(stage.at[pl.ds(0, WREM), :], out_hbm.at[cid, pl.ds(r0, WREM), :])


@functools.cache
def _sc_aggregate_kernel():
    return functools.partial(
        pl.kernel,
        out_type=jax.ShapeDtypeStruct((NC, N, DH), jnp.float32),
        mesh=_sc_mesh(),
        compiler_params=pltpu.CompilerParams(use_tc_tiling_on_sc=False),
        scratch_types=[
            pltpu.VMEM((K, CH), jnp.int32),
            pltpu.VMEM((K, CH), jnp.int32),
            pltpu.VMEM((KB, CH, DH), jnp.float32),
            pltpu.VMEM((KB, CH, DH), jnp.float32),
            pltpu.VMEM((CH, DH), jnp.float32),
            pltpu.VMEM_SHARED((NPAD, DH), jnp.float32),
            pltpu.SemaphoreType.DMA,
            pltpu.SemaphoreType.DMA,
        ],
    )(_sc_aggregate_body)


def _sc_aggregate_body(mpa_hbm, mpb_hbm, src2_hbm, dst2_hbm, out_hbm,
                       sidx, didx, rows_a, rows_b, stage, acc, sem, ssem):
    cid = lax.axis_index("c")
    sid = lax.axis_index("s")

    def zero_row(r, _):
        def zero_col(c, _):
            stage[r, pl.ds(c * 16, 16)] = jnp.zeros((16,), jnp.float32)
            return 0

        lax.fori_loop(0, DH // 16, zero_col, 0)
        return 0

    lax.fori_loop(0, CH, zero_row, 0)

    def zero_chunk(z, _):
        pltpu.sync_copy(stage.at[pl.ds(0, WCH), :],
                        acc.at[pl.ds(sid * LT + z * WCH, WCH), :])
        return 0

    lax.fori_loop(0, LT // WCH, zero_chunk, 0)
    pltpu.sync_copy(stage.at[pl.ds(0, WREM), :],
                    acc.at[pl.ds(sid * LT + (LT // WCH) * WCH, WREM), :])
    plsc.subcore_barrier()

    def make_pipeline(mp_hbm):
        NBODY = RPT // K

        def run():
            def body(t, _):
                row0 = sid * RPT + t * K
                pltpu.sync_copy(src2_hbm.at[pl.ds(row0, K), :], sidx)
                pltpu.sync_copy(dst2_hbm.at[pl.ds(row0, K), :], didx)
                ag = [
                    pltpu.async_copy(mp_hbm.at[sidx.at[k]], rows_a.at[k], sem)
                    for k in range(KB)
                ]
                for c in ag:
                    c.wait()
                a_sc = [
                    pltpu.async_copy(rows_a.at[k], acc.at[didx.at[k]], ssem, add=True)
                    for k in range(KB)
                ]
                bg = [
                    pltpu.async_copy(mp_hbm.at[sidx.at[KB + k]], rows_b.at[k], sem)
                    for k in range(KB)
                ]
                for c in bg:
                    c.wait()
                b_sc = [
                    pltpu.async_copy(rows_b.at[k], acc.at[didx.at[KB + k]], ssem, add=True)
                    for k in range(KB)
                ]
                for c in a_sc:
                    c.wait()
                for c in b_sc:
                    c.wait()
                return 0

            lax.fori_loop(0, NBODY, body, 0)

        return run

    @pl.when(cid == 0)
    def _():
        make_pipeline(mpa_hbm)()

    @pl.when(cid == 1)
    def _():
        make_pipeline(mpb_hbm)()

    plsc.subcore_barrier()

    nch = lax.select(sid == NS - 1, (N - (NS - 1) * LT) // WCH, LT // WCH)

    def write_chunk(z, _):
        r0 = sid * LT + z * WCH
        pltpu.sync_copy(acc.at[pl.ds(r0, WCH), :], stage.at[pl.ds(0, WCH), :])
        pltpu.sync_copy(stage.at[pl.ds(0, WCH), :], out_hbm.at[cid, pl.ds(r0, WCH), :])
        return 0

    lax.fori_loop(0, nch, write_chunk, 0)

    @pl.when(sid < NS - 1)
    def _():
        r0 = sid * LT + 6 * WCH
        pltpu.sync_copy(acc.at[pl.ds(r0, WREM), :], stage.at[pl.ds(0, WREM), :])
        pltpu.sync_copy(stage.at[pl.ds(0, WREM), :], out_hbm.at[cid, pl.ds(r0, WREM), :])


def _mm(x, w):
    def body(x_ref, w_ref, o_ref):
        o_ref[...] = jnp.dot(x_ref[...], w_ref[...], preferred_element_type=jnp.float32)

    return pl.pallas_call(
        body,
        grid=(N // BR,),
        in_specs=[
            pl.BlockSpec((BR, D), lambda i: (i, 0)),
            pl.BlockSpec((D, D), lambda i: (0, 0)),
        ],
        out_specs=pl.BlockSpec((BR, D), lambda i: (i, 0)),
        out_shape=jax.ShapeDtypeStruct((N, D), jnp.float32),
    )(x, w)


def _dinv_scale(degp, u1):
    def body(dg_ref, u_ref, dv_ref, ma_ref, mb_ref):
        dv = lax.rsqrt(dg_ref[0] + dg_ref[1] + 1.0)
        dv_ref[...] = dv
        m = u_ref[...] * dv[:, 0:1]
        ma_ref[...] = m[:, :DH]
        mb_ref[...] = m[:, DH:]

    return pl.pallas_call(
        body,
        grid=(N // BR,),
        in_specs=[
            pl.BlockSpec((NC, BR, DEGW), lambda i: (0, i, 0)),
            pl.BlockSpec((BR, D), lambda i: (i, 0)),
        ],
        out_specs=[
            pl.BlockSpec((BR, DEGW), lambda i: (i, 0)),
            pl.BlockSpec((BR, DH), lambda i: (i, 0)),
            pl.BlockSpec((BR, DH), lambda i: (i, 0)),
        ],
        out_shape=[
            jax.ShapeDtypeStruct((N, DEGW), jnp.float32),
            jax.ShapeDtypeStruct((N, DH), jnp.float32),
            jax.ShapeDtypeStruct((N, DH), jnp.float32),
        ],
    )(degp, u1)


def _mid(aggp, ma, mb, dinv, b, w):
    def body(a_ref, ma_ref, mb_ref, dv_ref, b_ref, w_ref, h_ref, mna_ref, mnb_ref):
        dv = dv_ref[:, 0:1]
        agg = jnp.concatenate([a_ref[0] + ma_ref[...], a_ref[1] + mb_ref[...]], axis=1)
        h = dv * agg + b_ref[...]
        h_ref[...] = h
        mn = jnp.dot(dv * h, w_ref[...], preferred_element_type=jnp.float32)
        mna_ref[...] = mn[:, :DH]
        mnb_ref[...] = mn[:, DH:]

    return pl.pallas_call(
        body,
        grid=(N // BR,),
        in_specs=[
            pl.BlockSpec((NC, BR, DH), lambda i: (0, i, 0)),
            pl.BlockSpec((BR, DH), lambda i: (i, 0)),
            pl.BlockSpec((BR, DH), lambda i: (i, 0)),
            pl.BlockSpec((BR, DEGW), lambda i: (i, 0)),
            pl.BlockSpec((1, D), lambda i: (0, 0)),
            pl.BlockSpec((D, D), lambda i: (0, 0)),
        ],
        out_specs=[
            pl.BlockSpec((BR, D), lambda i: (i, 0)),
            pl.BlockSpec((BR, DH), lambda i: (i, 0)),
            pl.BlockSpec((BR, DH), lambda i: (i, 0)),
        ],
        out_shape=[
            jax.ShapeDtypeStruct((N, D), jnp.float32),
            jax.ShapeDtypeStruct((N, DH), jnp.float32),
            jax.ShapeDtypeStruct((N, DH), jnp.float32),
        ],
    )(aggp, ma, mb, dinv, b, w)


def _final(x, hs, wp, bp):
    def body(x_ref, hs_ref, wp_ref, bp_ref, o_ref):
        acc = jnp.dot(x_ref[...], wp_ref[0:D], preferred_element_type=jnp.float32)
        acc = acc + jnp.dot(hs_ref[0], wp_ref[D:2 * D], preferred_element_type=jnp.float32)
        acc = acc + jnp.dot(hs_ref[1], wp_ref[2 * D:3 * D], preferred_element_type=jnp.float32)
        acc = acc + jnp.dot(hs_ref[2], wp_ref[3 * D:4 * D], preferred_element_type=jnp.float32)
        o_ref[...] = acc + bp_ref[...]

    return pl.pallas_call(
        body,
        grid=(N // BR,),
        in_specs=[
            pl.BlockSpec((BR, D), lambda i: (i, 0)),
            pl.BlockSpec((3, BR, D), lambda i: (0, i, 0)),
            pl.BlockSpec((CAT, D), lambda i: (0, 0)),
            pl.BlockSpec((1, D), lambda i: (0, 0)),
        ],
        out_specs=pl.BlockSpec((BR, D), lambda i: (i, 0)),
        out_shape=jax.ShapeDtypeStruct((N, D), jnp.float32),
    )(x, hs, wp, bp)


def kernel(x, edge_index, W1, b1, W2, b2, W3, b3, Wp, bp):
    pad = E2 - E
    pad_ar = jnp.arange(pad, dtype=jnp.int32)
    src = jnp.concatenate([edge_index[0], pad_ar % N])
    dst = jnp.concatenate([edge_index[1], N + pad_ar % (NPAD - N)])
    src2 = src.reshape(E2 // CH, CH)
    dst2 = dst.reshape(E2 // CH, CH)
    b1r = b1.reshape(1, D)
    b2r = b2.reshape(1, D)
    b3r = b3.reshape(1, D)
    bpr = bp.reshape(1, D)

    sc_degree = _sc_degree_kernel()
    sc_aggregate = _sc_aggregate_kernel()

    const = jnp.stack([jnp.ones((CH, DEGW), jnp.float32),
                       jnp.zeros((CH, DEGW), jnp.float32)])
    degp = sc_degree(const, dst2)
    u1 = _mm(x, W1)
    dinv, m1a, m1b = _dinv_scale(degp, u1)

    wnext = jnp.stack([W2, W3, jnp.zeros_like(W3)])
    bstack = jnp.stack([b1r, b2r, b3r])

    def step(carry, xs):
        ma, mb = carry
        wn, bk = xs
        a = sc_aggregate(ma, mb, src2, dst2)
        h, mna, mnb = _mid(a, ma, mb, dinv, bk, wn)
        return (mna, mnb), h

    _, hs = lax.scan(step, (m1a, m1b), (wnext, bstack))
    return _final(x, hs, Wp, bpr)
